# Initial kernel scaffold; baseline (speedup 1.0000x reference)
import jax
import jax.numpy as jnp
from jax.experimental import pallas as pl


def kernel(x, gate_w, gate_b, W1, b1, W2, b2):
    raise NotImplementedError("write your pallas kernel here")


# speedup vs baseline: 1.4360x; 1.4360x over previous
import functools

import jax
import jax.numpy as jnp
from jax.experimental import pallas as pl
from jax.experimental.pallas import tpu as pltpu


def _moe_kernel(x_ref, gw_ref, gb_ref, W1_ref, b1_ref, W2_ref, b2_ref,
                out_ref, scores_ref, comb_ref, *, E):
    e = pl.program_id(0)

    @pl.when(e == 0)
    def _gate():
        logits = jnp.dot(x_ref[...], gw_ref[...],
                         preferred_element_type=jnp.float32) + gb_ref[...]
        m = jnp.max(logits, axis=-1, keepdims=True)
        ex = jnp.exp(logits - m)
        p = ex / jnp.sum(ex, axis=-1, keepdims=True)
        scores_ref[...] = p
        iota = jax.lax.broadcasted_iota(jnp.int32, p.shape, 1)
        m1 = jnp.max(p, axis=-1, keepdims=True)
        a1 = jnp.min(jnp.where(p == m1, iota, E), axis=-1, keepdims=True)
        p2 = jnp.where(iota == a1, -jnp.inf, p)
        m2 = jnp.max(p2, axis=-1, keepdims=True)
        a2 = jnp.min(jnp.where(p2 == m2, iota, E), axis=-1, keepdims=True)
        keep = (iota == a1) | (iota == a2)
        comb_ref[...] = jnp.where(keep, p, 0.0)

    h = jnp.maximum(
        jnp.dot(x_ref[...], W1_ref[0], preferred_element_type=jnp.float32)
        + b1_ref[0], 0.0)
    y = jnp.dot(h, W2_ref[0], preferred_element_type=jnp.float32) + b2_ref[0]
    iota = jax.lax.broadcasted_iota(jnp.int32, comb_ref.shape, 1)
    col = jnp.sum(jnp.where(iota == e, comb_ref[...], 0.0), axis=1, keepdims=True)
    contrib = col * y

    @pl.when(e == 0)
    def _init():
        out_ref[...] = contrib

    @pl.when(e > 0)
    def _acc():
        out_ref[...] += contrib


def kernel(x, gate_w, gate_b, W1, b1, W2, b2):
    B, D = x.shape
    E = gate_w.shape[1]
    H = W1.shape[2]
    out, scores = pl.pallas_call(
        functools.partial(_moe_kernel, E=E),
        grid=(E,),
        in_specs=[
            pl.BlockSpec((B, D), lambda e: (0, 0)),
            pl.BlockSpec((D, E), lambda e: (0, 0)),
            pl.BlockSpec((1, E), lambda e: (0, 0)),
            pl.BlockSpec((1, D, H), lambda e: (e, 0, 0)),
            pl.BlockSpec((1, 1, H), lambda e: (e, 0, 0)),
            pl.BlockSpec((1, H, D), lambda e: (e, 0, 0)),
            pl.BlockSpec((1, 1, D), lambda e: (e, 0, 0)),
        ],
        out_specs=[
            pl.BlockSpec((B, D), lambda e: (0, 0)),
            pl.BlockSpec((B, E), lambda e: (0, 0)),
        ],
        out_shape=[
            jax.ShapeDtypeStruct((B, D), jnp.float32),
            jax.ShapeDtypeStruct((B, E), jnp.float32),
        ],
        scratch_shapes=[pltpu.VMEM((B, E), jnp.float32)],
    )(x, gate_w, gate_b.reshape(1, E), W1, b1.reshape(E, 1, H), W2,
      b2.reshape(E, 1, D))
    return (out, jax.lax.stop_gradient(scores))
